# Initial kernel scaffold; baseline (speedup 1.0000x reference)
import jax
import jax.numpy as jnp
from jax.experimental import pallas as pl


def kernel(x, edge_index, t, W1, b1, W2, b2, W3, b3):
    raise NotImplementedError("write your pallas kernel here")


# speedup vs baseline: 12.3561x; 12.3561x over previous
import functools

import jax
import jax.numpy as jnp
from jax import lax
from jax.experimental import pallas as pl
from jax.experimental.pallas import tpu as pltpu
from jax.experimental.pallas import tpu_sc as plsc

F = 128
NSC = 2
NSUB = 16
K = 125


def _mesh():
  return plsc.VectorSubcoreMesh(core_axis_name="c", subcore_axis_name="s")


def _sc_degree(dstd, zeros_f, ones_f, n_pad):
  nbd = dstd.shape[1]
  np_rows = n_pad // NSUB

  def body(dstd_h, zeros_h, ones_h, deg0_h, deg1_h, dstv, onesv, hist):
    cid = lax.axis_index("c")
    sid = lax.axis_index("s")
    w = cid * NSUB + sid
    pltpu.sync_copy(dstd_h.at[w], dstv)
    pltpu.sync_copy(ones_h, onesv)
    pltpu.sync_copy(zeros_h, hist.at[pl.ds(sid * np_rows, np_rows)])
    plsc.subcore_barrier()

    def batch(j, carry):
      pltpu.sync_copy(onesv, hist.at[dstv.at[j]], add=True)
      return carry

    lax.fori_loop(0, nbd, batch, 0)
    plsc.subcore_barrier()

    @pl.when(cid == 0)
    def _():
      pltpu.sync_copy(hist.at[pl.ds(sid * np_rows, np_rows)],
                      deg0_h.at[pl.ds(sid * np_rows, np_rows)])

    @pl.when(cid == 1)
    def _():
      pltpu.sync_copy(hist.at[pl.ds(sid * np_rows, np_rows)],
                      deg1_h.at[pl.ds(sid * np_rows, np_rows)])

  out_type = (jax.ShapeDtypeStruct((n_pad, F), jnp.float32),
              jax.ShapeDtypeStruct((n_pad, F), jnp.float32))
  scratch = [
      pltpu.VMEM((nbd, K), jnp.int32),
      pltpu.VMEM((K, F), jnp.float32),
      pltpu.VMEM_SHARED((n_pad, F), jnp.float32),
  ]
  return pl.kernel(body, out_type, mesh=_mesh(), scratch_types=scratch)(
      dstd, zeros_f, ones_f)


def _sc_spmm(p_chunks, src3, dst3, zeros_f, n_pad):
  c_total = len(p_chunks)
  assert c_total % NSC == 0
  ch = c_total // NSC
  nb = src3.shape[1]
  np_rows = n_pad // NSUB
  assert zeros_f.shape[0] == np_rows

  def body(*refs):
    p_refs = refs[:c_total]
    src3_h, dst3_h, zeros_h = refs[c_total:c_total + 3]
    u_refs = refs[c_total + 3:2 * c_total + 3]
    srcv, dstv, rows, acc = refs[2 * c_total + 3:]
    cid = lax.axis_index("c")
    sid = lax.axis_index("s")
    pltpu.sync_copy(src3_h.at[sid], srcv)
    pltpu.sync_copy(dst3_h.at[sid], dstv)

    def run_chunk(p_h):
      def batch(j, carry):
        pltpu.sync_copy(p_h.at[srcv.at[j]], rows)
        pltpu.sync_copy(rows, acc.at[dstv.at[j]], add=True)
        return carry
      lax.fori_loop(0, nb, batch, 0)

    def copy_out(u_h):
      pltpu.sync_copy(acc.at[pl.ds(sid * np_rows, np_rows)],
                      u_h.at[pl.ds(sid * np_rows, np_rows)])

    for i in range(ch):
      pltpu.sync_copy(zeros_h, acc.at[pl.ds(sid * np_rows, np_rows)])
      plsc.subcore_barrier()
      pl.when(cid == 0)(functools.partial(run_chunk, p_refs[i]))
      pl.when(cid == 1)(functools.partial(run_chunk, p_refs[ch + i]))
      plsc.subcore_barrier()
      pl.when(cid == 0)(functools.partial(copy_out, u_refs[i]))
      pl.when(cid == 1)(functools.partial(copy_out, u_refs[ch + i]))

  out_type = tuple(jax.ShapeDtypeStruct((n_pad, F), jnp.float32)
                   for _ in range(c_total))
  scratch = [
      pltpu.VMEM((nb, K), jnp.int32),
      pltpu.VMEM((nb, K), jnp.int32),
      pltpu.VMEM((K, F), jnp.float32),
      pltpu.VMEM_SHARED((n_pad, F), jnp.float32),
  ]
  return pl.kernel(body, out_type, mesh=_mesh(), scratch_types=scratch)(
      *p_chunks, src3, dst3, zeros_f)


def _block(bn, d):
  return pl.BlockSpec((bn, d), lambda i: (i, 0))


def _full(shape):
  return pl.BlockSpec(shape, lambda i: tuple(0 for _ in shape))


def _tc_prep(deg0, deg1, x, bn=1000):
  n, d = x.shape
  c = d // F

  def body(d0, d1, xr, dv_r, *p_refs):
    deg = d0[:, 0:1] + d1[:, 0:1] + 1.0
    dv = lax.rsqrt(deg)
    dv_r[...] = dv
    p = xr[...] * dv
    for i in range(c):
      p_refs[i][...] = p[:, i * F:(i + 1) * F]

  out_shape = ([jax.ShapeDtypeStruct((n, 1), jnp.float32)] +
               [jax.ShapeDtypeStruct((n, F), jnp.float32)] * c)
  return pl.pallas_call(
      body, grid=(n // bn,),
      in_specs=[_block(bn, F), _block(bn, F), _block(bn, d)],
      out_specs=[_block(bn, 1)] + [_block(bn, F)] * c,
      out_shape=out_shape)(deg0, deg1, x)


def _tc_layer(dinv, p_chunks, u_chunks, w, b, w_next=None, last_bias=None,
              bn=1000):
  n = dinv.shape[0]
  c_in = len(p_chunks)
  d_out = (w_next.shape[1] if w_next is not None else w.shape[1])
  c_out = d_out // F

  def body(*refs):
    dv_r = refs[0]
    p_refs = refs[1:1 + c_in]
    u_refs = refs[1 + c_in:1 + 2 * c_in]
    idx = 1 + 2 * c_in
    w_r = refs[idx]; idx += 1
    b_r = refs[idx] if b is not None else None
    idx += (b is not None)
    wn_r = refs[idx] if w_next is not None else None
    idx += (w_next is not None)
    lb_r = refs[idx] if last_bias is not None else None
    idx += (last_bias is not None)
    out_refs = refs[idx:]

    dv = dv_r[...]
    acc = None
    for i in range(c_in):
      s = (p_refs[i][...] + u_refs[i][...]) * dv
      part = jnp.dot(s, w_r[i * F:(i + 1) * F, :],
                     preferred_element_type=jnp.float32)
      acc = part if acc is None else acc + part
    if b is not None:
      acc = acc + b_r[...]
      h = jnp.tanh(acc)
    else:
      h = acc
    if w_next is not None:
      h = jnp.dot(h, wn_r[...], preferred_element_type=jnp.float32)
    if last_bias is not None:
      out_refs[0][...] = h + lb_r[...]
    else:
      h = h * dv
      for i in range(c_out):
        out_refs[i][...] = h[:, i * F:(i + 1) * F]

  in_specs = [_block(bn, 1)] + [_block(bn, F)] * (2 * c_in) + [_full(w.shape)]
  args = [dinv] + list(p_chunks) + list(u_chunks) + [w]
  if b is not None:
    in_specs.append(_full(b.shape)); args.append(b)
  if w_next is not None:
    in_specs.append(_full(w_next.shape)); args.append(w_next)
  if last_bias is not None:
    in_specs.append(_full(last_bias.shape)); args.append(last_bias)
    out_specs = [_block(bn, d_out)]
    out_shape = [jax.ShapeDtypeStruct((n, d_out), jnp.float32)]
  else:
    out_specs = [_block(bn, F)] * c_out
    out_shape = [jax.ShapeDtypeStruct((n, F), jnp.float32)] * c_out
  res = pl.pallas_call(body, grid=(n // bn,), in_specs=in_specs,
                       out_specs=out_specs, out_shape=out_shape)(*args)
  return res if last_bias is None else res[0]


def _tc_final(dinv, p_chunks, u_chunks, b, bn=1000):
  n = dinv.shape[0]
  c = len(p_chunks)
  d = c * F

  def body(*refs):
    dv_r = refs[0]
    p_refs = refs[1:1 + c]
    u_refs = refs[1 + c:1 + 2 * c]
    b_r = refs[1 + 2 * c]
    out_r = refs[2 + 2 * c]
    dv = dv_r[...]
    parts = [(p_refs[i][...] + u_refs[i][...]) * dv for i in range(c)]
    out_r[...] = jnp.concatenate(parts, axis=1) + b_r[...]

  in_specs = ([_block(bn, 1)] + [_block(bn, F)] * (2 * c) + [_full(b.shape)])
  return pl.pallas_call(
      body, grid=(n // bn,), in_specs=in_specs,
      out_specs=[_block(bn, d)],
      out_shape=[jax.ShapeDtypeStruct((n, d), jnp.float32)])(
          dinv, *p_chunks, *u_chunks, b)[0]


def kernel(x, edge_index, t, W1, b1, W2, b2, W3, b3):
  n, d_in = x.shape
  e = edge_index.shape[1]
  ep = e // NSUB
  nb = ep // K
  epd = e // (NSC * NSUB)
  nbd = epd // K
  assert ep % K == 0 and epd % K == 0
  n_pad = NSUB * 128 * ((n + NSUB * 128 - 1) // (NSUB * 128))

  src = edge_index[0]
  dst = edge_index[1]
  src3 = src.reshape(NSUB, nb, K)
  dst3 = dst.reshape(NSUB, nb, K)
  dstd = dst.reshape(NSC * NSUB, nbd, K)

  zeros_f = jnp.zeros((n_pad // NSUB, F), jnp.float32)
  ones_f = jnp.ones((K, F), jnp.float32)

  deg0, deg1 = _sc_degree(dstd, zeros_f, ones_f, n_pad)
  prep = _tc_prep(deg0, deg1, x)
  dinv, p0 = prep[0], tuple(prep[1:])

  u0 = _sc_spmm(p0, src3, dst3, zeros_f, n_pad)
  p1 = _tc_layer(dinv, p0, u0, W1, b1.reshape(1, -1))
  u1 = _sc_spmm(tuple(p1), src3, dst3, zeros_f, n_pad)
  p2 = _tc_layer(dinv, p1, u1, W2, b2.reshape(1, -1), w_next=W3)
  u2 = _sc_spmm(tuple(p2), src3, dst3, zeros_f, n_pad)
  out = _tc_final(dinv, p2, u2, b3.reshape(1, -1))
  return out
